# Initial kernel scaffold; baseline (speedup 1.0000x reference)
import jax
import jax.numpy as jnp
from jax.experimental import pallas as pl


def kernel(x):
    raise NotImplementedError("write your pallas kernel here")


# speedup vs baseline: 2.2839x; 2.2839x over previous
import functools

import jax
import jax.numpy as jnp
import numpy as np
from jax import lax
from jax.experimental import pallas as pl
from jax.experimental.pallas import tpu as pltpu
from jax.experimental.pallas import tpu_sc as plsc

_ROWS = 128
_N = 32768
_L = 16
_NV = _N // _L
_RADIX = 2048
_SHIFTS = (0, 11, 22)
_MASKS = (0x7FF, 0x7FF, 0x3FF)
_NC = 2
_NS = 16
_ROWS_PER_W = _ROWS // (_NC * _NS)
_MININT = np.int32(-2147483648)


def _to_sortable(u):
    return u ^ (jnp.right_shift(u, 31) | _MININT)


def _from_sortable(u):
    return u ^ (jnp.right_shift(~u, 31) | _MININT)


def _digit(u, shift, mask):
    ub = plsc.bitcast(u, jnp.uint32)
    return ((ub >> shift) & jnp.uint32(mask)).astype(jnp.int32)


def _sort_body(x_hbm, out_hbm, a_v, b_v, hist_v):
    wid = lax.axis_index("s") * _NC + lax.axis_index("c")

    def zero_hist(j, _):
        hist_v[pl.ds(j * _L, _L)] = jnp.zeros((_L,), jnp.int32)
        return 0

    def prefix(j, carry):
        v = hist_v[pl.ds(j * _L, _L)]
        c = plsc.cumsum(v)
        hist_v[pl.ds(j * _L, _L)] = c - v + carry
        return carry + jnp.sum(v)

    def do_row(r, _):
        row = wid * _ROWS_PER_W + r
        pltpu.sync_copy(x_hbm.at[row], a_v)

        for p in range(3):
            src, dst = (a_v, b_v) if p % 2 == 0 else (b_v, a_v)
            shift, mask = _SHIFTS[p], _MASKS[p]

            lax.fori_loop(0, _RADIX // _L, zero_hist, 0, unroll=4)

            if p == 0:
                def hist0(i, _):
                    sl = pl.ds(i * _L, _L)
                    u = plsc.bitcast(src[sl], jnp.int32)
                    u = _to_sortable(u)
                    src[sl] = plsc.bitcast(u, jnp.float32)
                    d = _digit(u, shift, mask)
                    cnt, last = plsc.scan_count(d)
                    plsc.addupdate_scatter(hist_v, [d], cnt, mask=last)
                    return 0

                lax.fori_loop(0, _NV, hist0, 0, unroll=2)
            else:
                def hist(i, _):
                    u = plsc.bitcast(src[pl.ds(i * _L, _L)], jnp.int32)
                    d = _digit(u, shift, mask)
                    cnt, last = plsc.scan_count(d)
                    plsc.addupdate_scatter(hist_v, [d], cnt, mask=last)
                    return 0

                lax.fori_loop(0, _NV, hist, 0, unroll=2)

            lax.fori_loop(0, _RADIX // _L, prefix, jnp.int32(0), unroll=2)

            def permute(i, _):
                u = plsc.bitcast(src[pl.ds(i * _L, _L)], jnp.int32)
                d = _digit(u, shift, mask)
                cnt, last = plsc.scan_count(d)
                base = plsc.load_gather(hist_v, [d])
                pos = base + cnt - 1
                if p == 2:
                    u = _from_sortable(u)
                plsc.store_scatter(dst, [pos], plsc.bitcast(u, jnp.float32))
                plsc.store_scatter(hist_v, [d], base + cnt, mask=last)
                return 0

            lax.fori_loop(0, _NV, permute, 0, unroll=2)

        pltpu.sync_copy(b_v, out_hbm.at[row])
        return 0

    lax.fori_loop(0, _ROWS_PER_W, do_row, 0)


@jax.jit
def kernel(x):
    mesh = plsc.VectorSubcoreMesh(
        core_axis_name="c", subcore_axis_name="s", num_cores=_NC,
        num_subcores=_NS)
    run = pl.kernel(
        _sort_body,
        out_type=jax.ShapeDtypeStruct((_ROWS, _N), jnp.float32),
        mesh=mesh,
        scratch_types=[
            pltpu.VMEM((_N,), jnp.float32),
            pltpu.VMEM((_N,), jnp.float32),
            pltpu.VMEM((_RADIX,), jnp.int32),
        ],
        compiler_params=pltpu.CompilerParams(needs_layout_passes=False),
    )
    return run(x)


# speedup vs baseline: 3.2528x; 1.4243x over previous
import functools

import jax
import jax.numpy as jnp
import numpy as np
from jax import lax
from jax.experimental import pallas as pl
from jax.experimental.pallas import tpu as pltpu
from jax.experimental.pallas import tpu_sc as plsc

_ROWS = 128
_N = 32768
_L = 16
_NV = _N // _L
_SHIFTS = (0, 11, 22)
_MASKS = (0x7FF, 0x7FF, 0x3FF)
_RSIZE = (2048, 2048, 1024)
_NC = 2
_NS = 16
_ROWS_PER_W = _ROWS // (_NC * _NS)
_MININT = np.int32(-2147483648)


def _to_sortable(u):
    return u ^ (jnp.right_shift(u, 31) | _MININT)


def _from_sortable(u):
    return u ^ (jnp.right_shift(~u, 31) | _MININT)


def _digit(u, shift, mask):
    ub = plsc.bitcast(u, jnp.uint32)
    return ((ub >> shift) & jnp.uint32(mask)).astype(jnp.int32)


def _last_lane(v):
    return lax.squeeze(lax.slice(v, (_L - 1,), (_L,)), (0,))


def _sort_body(x_hbm, out_hbm, a_v, b_v, h0_v, h1_v, h2_v):
    wid = lax.axis_index("s") * _NC + lax.axis_index("c")
    hists = (h0_v, h1_v, h2_v)
    zeros = jnp.zeros((_L,), jnp.int32)

    def do_row(r, _):
        row = wid * _ROWS_PER_W + r
        pltpu.sync_copy(x_hbm.at[row], a_v)

        def zero_all(j, _):
            sl = pl.ds(j * _L, _L)
            h0_v[sl] = zeros
            h1_v[sl] = zeros
            return 0

        lax.fori_loop(0, 2048 // _L, zero_all, 0, unroll=8)

        def zero_h2(j, _):
            h2_v[pl.ds(j * _L, _L)] = zeros
            return 0

        lax.fori_loop(0, 1024 // _L, zero_h2, 0, unroll=8)

        def hist_all(i, _):
            sl = pl.ds(i * _L, _L)
            u = plsc.bitcast(a_v[sl], jnp.int32)
            u = _to_sortable(u)
            a_v[sl] = plsc.bitcast(u, jnp.float32)
            for p in range(3):
                d = _digit(u, _SHIFTS[p], _MASKS[p])
                cnt, last = plsc.scan_count(d)
                plsc.addupdate_scatter(hists[p], [d], cnt, mask=last)
            return 0

        lax.fori_loop(0, _NV, hist_all, 0, unroll=2)

        for p in range(3):
            src, dst = (a_v, b_v) if p % 2 == 0 else (b_v, a_v)
            shift, mask, hist = _SHIFTS[p], _MASKS[p], hists[p]

            def prefix(j, carry, hist=hist):
                sl = pl.ds(j * _L, _L)
                v = hist[sl]
                c = plsc.cumsum(v)
                hist[sl] = c - v + carry
                return carry + _last_lane(c)

            lax.fori_loop(0, _RSIZE[p] // _L, prefix, jnp.int32(0),
                          unroll=2)

            def permute(i, _, src=src, dst=dst, shift=shift, mask=mask,
                        hist=hist, p=p):
                u = plsc.bitcast(src[pl.ds(i * _L, _L)], jnp.int32)
                d = _digit(u, shift, mask)
                cnt, last = plsc.scan_count(d)
                base = plsc.load_gather(hist, [d])
                pos = base + cnt - 1
                if p == 2:
                    u = _from_sortable(u)
                plsc.store_scatter(dst, [pos], plsc.bitcast(u, jnp.float32))
                plsc.store_scatter(hist, [d], base + cnt, mask=last)
                return 0

            lax.fori_loop(0, _NV, permute, 0, unroll=4)

        pltpu.sync_copy(b_v, out_hbm.at[row])
        return 0

    lax.fori_loop(0, _ROWS_PER_W, do_row, 0)


@jax.jit
def kernel(x):
    mesh = plsc.VectorSubcoreMesh(
        core_axis_name="c", subcore_axis_name="s", num_cores=_NC,
        num_subcores=_NS)
    run = pl.kernel(
        _sort_body,
        out_type=jax.ShapeDtypeStruct((_ROWS, _N), jnp.float32),
        mesh=mesh,
        scratch_types=[
            pltpu.VMEM((_N,), jnp.float32),
            pltpu.VMEM((_N,), jnp.float32),
            pltpu.VMEM((2048,), jnp.int32),
            pltpu.VMEM((2048,), jnp.int32),
            pltpu.VMEM((1024,), jnp.int32),
        ],
        compiler_params=pltpu.CompilerParams(needs_layout_passes=False),
    )
    return run(x)
